# Initial kernel scaffold; baseline (speedup 1.0000x reference)
import jax
import jax.numpy as jnp
from jax.experimental import pallas as pl


def kernel(hidden_states, Wq, keys, down_embed, up_embed, Wg, Wu, Wd):
    raise NotImplementedError("write your pallas kernel here")


# speedup vs baseline: 1.7353x; 1.7353x over previous
import jax
import jax.numpy as jnp
from jax.experimental import pallas as pl
from jax.experimental.pallas import tpu as pltpu

_B, _S, _HID = 2, 2048, 2048
_INTER = 5504
_RET = 128
_NE = 64
_TOPK = 8
_NK = 8

_INTER_PAD = 5632
_TN = 1024
_TK = 256


def _route_proj_kernel(h0_ref, h1_ref, wq_ref, keys_ref, r0_ref, r1_ref):
    dn = (((0,), (0,)), ((), ()))
    wq = wq_ref[...]
    k0 = keys_ref[0:64, :]
    k1 = keys_ref[64:128, :]
    p0a = jax.lax.dot_general(wq[0:64, :], k0, dn, preferred_element_type=jnp.float32)
    p0b = jax.lax.dot_general(wq[64:128, :], k0, dn, preferred_element_type=jnp.float32)
    p1a = jax.lax.dot_general(wq[0:64, :], k1, dn, preferred_element_type=jnp.float32)
    p1b = jax.lax.dot_general(wq[64:128, :], k1, dn, preferred_element_type=jnp.float32)
    P0 = jnp.concatenate([p0a, p0b], axis=1)
    P1 = jnp.concatenate([p1a, p1b], axis=1)
    r0_ref[...] = jnp.dot(h0_ref[...], P0, preferred_element_type=jnp.float32)
    r1_ref[...] = jnp.dot(h1_ref[...], P1, preferred_element_type=jnp.float32)


def _moe_mlp_kernel(rw0_ref, rw1_ref, h_ref, down_ref, up_ref,
                    wg_ref, wu_ref, wd_ref, out_ref, w64_ref):
    k = pl.program_id(1)

    @pl.when(k == 0)
    def _routing():
        rw0 = rw0_ref[...]
        rw1 = rw1_ref[...]
        col = jax.lax.broadcasted_iota(jnp.int32, (8, 64), 1)
        row = jax.lax.broadcasted_iota(jnp.int32, (8, 64), 0)
        e1 = (col // 8 == row).astype(jnp.float32)
        e2 = (col % 8 == row).astype(jnp.float32)
        s64 = (jnp.dot(rw0, e1, preferred_element_type=jnp.float32)
               + jnp.dot(rw1, e2, preferred_element_type=jnp.float32))
        cur = s64
        m0 = jnp.max(cur, axis=1, keepdims=True)
        m = m0
        for _ in range(_TOPK - 1):
            cur = jnp.where(cur >= m, -jnp.inf, cur)
            m = jnp.max(cur, axis=1, keepdims=True)
        mask = s64 >= m
        p = jnp.where(mask, jnp.exp(s64 - m0), 0.0)
        p = p / jnp.sum(p, axis=1, keepdims=True)
        L = jax.lax.dot_general(h_ref[...], down_ref[...],
                                (((1,), (1,)), ((), ())),
                                preferred_element_type=jnp.float32)
        w64_ref[...] = L * jax.nn.sigmoid(L) * p

    dnT = (((1,), (1,)), ((), ()))
    h = h_ref[...]
    g = jax.lax.dot_general(h, wg_ref[...], dnT, preferred_element_type=jnp.float32)
    u = jax.lax.dot_general(h, wu_ref[...], dnT, preferred_element_type=jnp.float32)
    a = g * jax.nn.sigmoid(g) * u
    part = jax.lax.dot_general(a, wd_ref[...], dnT, preferred_element_type=jnp.float32)

    @pl.when(k == 0)
    def _first():
        out_ref[...] = part + jnp.dot(w64_ref[...], up_ref[...],
                                      preferred_element_type=jnp.float32)

    @pl.when(k > 0)
    def _acc():
        out_ref[...] += part


def kernel(hidden_states, Wq, keys, down_embed, up_embed, Wg, Wu, Wd):
    b, s, h = hidden_states.shape
    N = b * s
    hflat = hidden_states.reshape(N, h)
    keys2 = keys.reshape(2 * (_RET // 2), _NK)

    r0, r1 = pl.pallas_call(
        _route_proj_kernel,
        grid=(2,),
        in_specs=[
            pl.BlockSpec((s // 2, h), lambda i: (i, 0)),
            pl.BlockSpec((s // 2, h), lambda i: (i, 0)),
            pl.BlockSpec((_RET, h), lambda i: (0, 0)),
            pl.BlockSpec((2 * (_RET // 2), _NK), lambda i: (0, 0)),
        ],
        out_specs=[
            pl.BlockSpec((s // 2, 16), lambda i: (i, 0)),
            pl.BlockSpec((s // 2, 16), lambda i: (i, 0)),
        ],
        out_shape=[
            jax.ShapeDtypeStruct((s, 16), jnp.float32),
            jax.ShapeDtypeStruct((s, 16), jnp.float32),
        ],
    )(hidden_states[0], hidden_states[1], Wq, keys2)

    rw0 = r0.reshape(N, _NK)
    rw1 = r1.reshape(N, _NK)

    wg_p = jnp.pad(Wg, ((0, _INTER_PAD - _INTER), (0, 0)))
    wu_p = jnp.pad(Wu, ((0, _INTER_PAD - _INTER), (0, 0)))
    wd_p = jnp.pad(Wd, ((0, 0), (0, _INTER_PAD - _INTER)))

    nt = N // _TN
    kt = _INTER_PAD // _TK
    out = pl.pallas_call(
        _moe_mlp_kernel,
        grid=(nt, kt),
        in_specs=[
            pl.BlockSpec((_TN, _NK), lambda n, k: (n, 0)),
            pl.BlockSpec((_TN, _NK), lambda n, k: (n, 0)),
            pl.BlockSpec((_TN, h), lambda n, k: (n, 0)),
            pl.BlockSpec((_NE, h), lambda n, k: (0, 0)),
            pl.BlockSpec((_NE, h), lambda n, k: (0, 0)),
            pl.BlockSpec((_TK, h), lambda n, k: (k, 0)),
            pl.BlockSpec((_TK, h), lambda n, k: (k, 0)),
            pl.BlockSpec((h, _TK), lambda n, k: (0, k)),
        ],
        out_specs=pl.BlockSpec((_TN, h), lambda n, k: (n, 0)),
        out_shape=jax.ShapeDtypeStruct((N, h), jnp.float32),
        scratch_shapes=[pltpu.VMEM((_TN, _NE), jnp.float32)],
        compiler_params=pltpu.CompilerParams(
            dimension_semantics=("arbitrary", "arbitrary"),
        ),
    )(rw0, rw1, hflat, down_embed, up_embed, wg_p, wu_p, wd_p)

    return out.reshape(b, s, h)
